# baseline (speedup 1.0000x reference)
import functools

import jax
import jax.numpy as jnp
from jax import lax
from jax.experimental import pallas as pl
from jax.experimental.pallas import tpu as pltpu
from jax.experimental.pallas import tpu_sc as plsc

N = 32768
H = 768
L = 16
HV = H // L
NC, NS = 2, 16
NW = NC * NS
NPW = N // NW
CT = 8
G = 9
GS = G * CT
NCH = NPW // CT
NQ = NCH // 2
INV_H = 1.0 / H
EPS = 1e-12

_SIZES = (1024, 128, 1024, 1024, 1024, 1024)
_OFF = []
_acc = 0
for _s in _SIZES:
    _OFF.append(_acc)
    _acc += _s


def _rsqrt(x):
    xi = lax.bitcast_convert_type(x, jnp.int32)
    y = lax.bitcast_convert_type(0x5F3759DF - (xi >> 1), jnp.float32)
    for _ in range(3):
        y = y * (1.5 - 0.5 * x * y * y)
    return y


def _reduce_splat(v):
    dnums = lax.GatherDimensionNumbers(
        offset_dims=(), collapsed_slice_dims=(0,), start_index_map=(0,))
    for off in (8, 4, 2, 1):
        perm = lax.iota(jnp.int32, L) ^ off
        v = v + lax.gather(v, perm[:, None], dnums, (1,),
                           mode=lax.GatherScatterMode.PROMISE_IN_BOUNDS)
    return v


def _body(word_hbm, small_hbm, idx_hbm, gb_hbm, out_hbm,
          idx_v, buf_a, buf_b, obuf, gb_v, s_a, s_b, s_o):
    wid = lax.axis_index("s") * NC + lax.axis_index("c")
    base = wid * NPW
    pltpu.sync_copy(gb_hbm, gb_v)
    pltpu.sync_copy(idx_hbm.at[pl.ds(wid * (NCH * GS), NCH * GS)], idx_v)

    def _descs(c, buf, sem):
        small = pltpu.make_async_copy(
            small_hbm.at[idx_v.at[pl.ds(c * GS + CT, (G - 1) * CT)]],
            buf.at[pl.ds(0, (G - 1) * CT)], sem)
        word = pltpu.make_async_copy(
            word_hbm.at[idx_v.at[pl.ds(c * GS, CT)]],
            buf.at[pl.ds((G - 1) * CT, CT)], sem)
        return small, word

    def gather(c, buf, sem):
        for d in _descs(c, buf, sem):
            d.start()

    def gather_wait(c, buf, sem):
        for d in _descs(c, buf, sem):
            d.wait()

    def out_desc(c):
        return pltpu.make_async_copy(
            obuf, out_hbm.at[pl.ds(base + c * CT, CT)], s_o)

    U = 3

    def token(buf, t):
        rows = [(G - 1) * CT + t] + [k * CT + t for k in range(G - 1)]

        def p1(ii, carry):
            s, q = carry
            for u in range(U):
                sl = pl.ds((ii * U + u) * L, L)
                vs = [buf[r, sl] for r in rows]
                while len(vs) > 1:
                    vs = [a + b for a, b in zip(vs[::2], vs[1::2])] + (
                        [vs[-1]] if len(vs) % 2 else [])
                v = vs[0]
                obuf[t, sl] = v
                s = s + v
                q = q + v * v
            return s, q
        z = jnp.zeros((L,), jnp.float32)
        s, q = lax.fori_loop(0, HV // U, p1, (z, z))
        mu = _reduce_splat(s) * INV_H
        var = _reduce_splat(q) * INV_H - mu * mu
        rstd = _rsqrt(var + EPS)

        def p2(ii, _):
            for u in range(U):
                sl = pl.ds((ii * U + u) * L, L)
                t1 = rstd * gb_v[0, sl]
                t2 = gb_v[1, sl] - mu * t1
                obuf[t, sl] = obuf[t, sl] * t1 + t2
            return 0
        lax.fori_loop(0, HV // U, p2, 0)

    def chunk(buf):
        for t in range(CT):
            token(buf, t)

    def body(q, _):
        c0 = 2 * q
        gather_wait(c0, buf_a, s_a)

        @pl.when(q >= 1)
        def _():
            out_desc(c0 - 1).wait()
        chunk(buf_a)
        out_desc(c0).start()

        @pl.when(q < NQ - 1)
        def _():
            gather(c0 + 2, buf_a, s_a)
        gather_wait(c0 + 1, buf_b, s_b)
        out_desc(c0).wait()
        chunk(buf_b)
        out_desc(c0 + 1).start()

        @pl.when(q < NQ - 1)
        def _():
            gather(c0 + 3, buf_b, s_b)
        return 0

    gather(0, buf_a, s_a)
    gather(1, buf_b, s_b)
    lax.fori_loop(0, NQ, body, 0)
    out_desc(NCH - 1).wait()


@functools.cache
def _build():
    mesh = plsc.VectorSubcoreMesh(core_axis_name="c", subcore_axis_name="s",
                                  num_cores=NC, num_subcores=NS)
    return pl.kernel(
        _body,
        out_type=jax.ShapeDtypeStruct((N, H), jnp.float32),
        mesh=mesh,
        scratch_types=[
            pltpu.VMEM((NCH * GS,), jnp.int32),
            pltpu.VMEM((GS, H), jnp.float32),
            pltpu.VMEM((GS, H), jnp.float32),
            pltpu.VMEM((CT, H), jnp.float32),
            pltpu.VMEM((2, H), jnp.float32),
            pltpu.SemaphoreType.DMA,
            pltpu.SemaphoreType.DMA,
            pltpu.SemaphoreType.DMA,
        ],
    )


def kernel(input_ids, bbox, token_type_ids, position_ids, font_ids,
           word_emb, pos_emb, x_emb, y_emb, h_emb, w_emb, tok_emb, font_emb,
           gamma, beta):
    B, S = input_ids.shape
    i32 = jnp.int32
    postok = (tok_emb[:, None, :] + pos_emb[None, :, :]).reshape(-1, H)
    small = jnp.concatenate([postok, font_emb, x_emb, y_emb, h_emb, w_emb],
                            axis=0)
    ids = input_ids.reshape(N).astype(i32)
    pos_idx = jnp.broadcast_to(position_ids, (B, S)).reshape(N).astype(i32)
    tok_idx = token_type_ids.reshape(N).astype(i32)
    font_idx = font_ids.reshape(N).astype(i32)
    bb = bbox.astype(i32)
    left = bb[:, :, 0].reshape(N)
    upper = bb[:, :, 1].reshape(N)
    right = bb[:, :, 2].reshape(N)
    lower = bb[:, :, 3].reshape(N)
    idx8 = jnp.stack([
        tok_idx * 512 + pos_idx + _OFF[0],
        font_idx + _OFF[1],
        left + _OFF[2],
        upper + _OFF[3],
        right + _OFF[2],
        lower + _OFF[3],
        (lower - upper) + _OFF[4],
        (right - left) + _OFF[5],
    ])
    idw = ids.reshape(NW, NCH, 1, CT)
    ism = idx8.reshape(8, NW, NCH, CT).transpose(1, 2, 0, 3)
    idx = jnp.concatenate([idw, ism], axis=2).reshape(-1)
    gb = jnp.stack([gamma, beta])
    out = _build()(word_emb, small, idx, gb)
    return out.reshape(B, S, H)


# speedup vs baseline: 2.1402x; 2.1402x over previous
import functools

import jax
import jax.numpy as jnp
from jax import lax
from jax.experimental import pallas as pl
from jax.experimental.pallas import tpu as pltpu
from jax.experimental.pallas import tpu_sc as plsc

N = 32768
H = 768
L = 16
HV = H // L
NC, NS = 2, 16
NW = NC * NS
NPW = N // NW
CT = 8
G = 9
GS = G * CT
NCH = NPW // CT
NQ = NCH // 2
INV_H = 1.0 / H
EPS = 1e-12

_SIZES = (1024, 128, 1024, 1024, 1024, 1024)
_OFF = []
_acc = 0
for _s in _SIZES:
    _OFF.append(_acc)
    _acc += _s


def _rsqrt(x):
    xi = lax.bitcast_convert_type(x, jnp.int32)
    y = lax.bitcast_convert_type(0x5F3759DF - (xi >> 1), jnp.float32)
    for _ in range(3):
        y = y * (1.5 - 0.5 * x * y * y)
    return y


def _reduce_splat(v):
    dnums = lax.GatherDimensionNumbers(
        offset_dims=(), collapsed_slice_dims=(0,), start_index_map=(0,))
    for off in (8, 4, 2, 1):
        perm = lax.iota(jnp.int32, L) ^ off
        v = v + lax.gather(v, perm[:, None], dnums, (1,),
                           mode=lax.GatherScatterMode.PROMISE_IN_BOUNDS)
    return v


def _body(word_hbm, small_hbm, idx_hbm, gb_hbm, out_hbm,
          idx_v, buf_a, buf_b, obuf, gb_v, s_a, s_b, s_o):
    wid = lax.axis_index("s") * NC + lax.axis_index("c")
    base = wid * NPW
    pltpu.sync_copy(gb_hbm, gb_v)
    pltpu.sync_copy(idx_hbm.at[pl.ds(wid * (NCH * GS), NCH * GS)], idx_v)

    def _descs(c, buf, sem):
        small = pltpu.make_async_copy(
            small_hbm.at[idx_v.at[pl.ds(c * GS + CT, (G - 1) * CT)]],
            buf.at[pl.ds(0, (G - 1) * CT)], sem)
        word = pltpu.make_async_copy(
            word_hbm.at[idx_v.at[pl.ds(c * GS, CT)]],
            buf.at[pl.ds((G - 1) * CT, CT)], sem)
        return small, word

    def gather(c, buf, sem):
        for d in _descs(c, buf, sem):
            d.start()

    def gather_wait(c, buf, sem):
        for d in _descs(c, buf, sem):
            d.wait()

    def out_desc(c):
        return pltpu.make_async_copy(
            obuf, out_hbm.at[pl.ds(base + c * CT, CT)], s_o)

    def token(buf, t):
        rows = [(G - 1) * CT + t] + [k * CT + t for k in range(G - 1)]
        z = jnp.zeros((L,), jnp.float32)

        def p1(i, carry):
            s, q = carry
            sl = pl.ds(i * L, L)
            vs = [buf[r, sl] for r in rows]
            while len(vs) > 1:
                vs = [a + b for a, b in zip(vs[::2], vs[1::2])] + (
                    [vs[-1]] if len(vs) % 2 else [])
            v = vs[0]
            obuf[t, sl] = v
            return s + v, q + v * v
        s, q = plsc.parallel_loop(0, HV, carry=(z, z), unroll=4)(p1)
        mu = _reduce_splat(s) * INV_H
        var = _reduce_splat(q) * INV_H - mu * mu
        rstd = _rsqrt(var + EPS)

        def p2(i):
            sl = pl.ds(i * L, L)
            t1 = rstd * gb_v[0, sl]
            t2 = gb_v[1, sl] - mu * t1
            obuf[t, sl] = obuf[t, sl] * t1 + t2
        plsc.parallel_loop(0, HV, unroll=4)(p2)

    def chunk(buf):
        for t in range(CT):
            token(buf, t)

    def body(q, _):
        c0 = 2 * q
        gather_wait(c0, buf_a, s_a)

        @pl.when(q >= 1)
        def _():
            out_desc(c0 - 1).wait()
        chunk(buf_a)
        out_desc(c0).start()

        @pl.when(q < NQ - 1)
        def _():
            gather(c0 + 2, buf_a, s_a)
        gather_wait(c0 + 1, buf_b, s_b)
        out_desc(c0).wait()
        chunk(buf_b)
        out_desc(c0 + 1).start()

        @pl.when(q < NQ - 1)
        def _():
            gather(c0 + 3, buf_b, s_b)
        return 0

    gather(0, buf_a, s_a)
    gather(1, buf_b, s_b)
    lax.fori_loop(0, NQ, body, 0)
    out_desc(NCH - 1).wait()


@functools.cache
def _build():
    mesh = plsc.VectorSubcoreMesh(core_axis_name="c", subcore_axis_name="s",
                                  num_cores=NC, num_subcores=NS)
    return pl.kernel(
        _body,
        out_type=jax.ShapeDtypeStruct((N, H), jnp.float32),
        mesh=mesh,
        scratch_types=[
            pltpu.VMEM((NCH * GS,), jnp.int32),
            pltpu.VMEM((GS, H), jnp.float32),
            pltpu.VMEM((GS, H), jnp.float32),
            pltpu.VMEM((CT, H), jnp.float32),
            pltpu.VMEM((2, H), jnp.float32),
            pltpu.SemaphoreType.DMA,
            pltpu.SemaphoreType.DMA,
            pltpu.SemaphoreType.DMA,
        ],
    )


def kernel(input_ids, bbox, token_type_ids, position_ids, font_ids,
           word_emb, pos_emb, x_emb, y_emb, h_emb, w_emb, tok_emb, font_emb,
           gamma, beta):
    B, S = input_ids.shape
    i32 = jnp.int32
    postok = (tok_emb[:, None, :] + pos_emb[None, :, :]).reshape(-1, H)
    small = jnp.concatenate([postok, font_emb, x_emb, y_emb, h_emb, w_emb],
                            axis=0)
    ids = input_ids.reshape(N).astype(i32)
    pos_idx = jnp.broadcast_to(position_ids, (B, S)).reshape(N).astype(i32)
    tok_idx = token_type_ids.reshape(N).astype(i32)
    font_idx = font_ids.reshape(N).astype(i32)
    bb = bbox.astype(i32)
    left = bb[:, :, 0].reshape(N)
    upper = bb[:, :, 1].reshape(N)
    right = bb[:, :, 2].reshape(N)
    lower = bb[:, :, 3].reshape(N)
    idx8 = jnp.stack([
        tok_idx * 512 + pos_idx + _OFF[0],
        font_idx + _OFF[1],
        left + _OFF[2],
        upper + _OFF[3],
        right + _OFF[2],
        lower + _OFF[3],
        (lower - upper) + _OFF[4],
        (right - left) + _OFF[5],
    ])
    idw = ids.reshape(NW, NCH, 1, CT)
    ism = idx8.reshape(8, NW, NCH, CT).transpose(1, 2, 0, 3)
    idx = jnp.concatenate([idw, ism], axis=2).reshape(-1)
    gb = jnp.stack([gamma, beta])
    out = _build()(word_emb, small, idx, gb)
    return out.reshape(B, S, H)
